# Initial kernel scaffold; baseline (speedup 1.0000x reference)
import jax
import jax.numpy as jnp
from jax.experimental import pallas as pl


def kernel(x, weight):
    raise NotImplementedError("write your pallas kernel here")


# speedup vs baseline: 3394.5690x; 3394.5690x over previous
import jax
import jax.numpy as jnp
from jax.experimental import pallas as pl
from jax.experimental.pallas import tpu as pltpu

_D = 2048
_ROWS = 4 * 4096
_R = 1024
_NBLK = _ROWS // _R


def _absmax_body(x_ref, o_ref):
    o_ref[...] = jnp.max(jnp.abs(x_ref[...])).reshape(1, 1, 1)


def _quant_stats_body(x_ref, w_ref, s_ref, xi8_ref, inv_ref, ymax_ref):
    s = s_ref[0, 0]
    x = x_ref[...]
    xi = jnp.clip(jnp.round(x / s), -127.0, 127.0)
    xi8_ref[...] = xi.astype(jnp.int8)

    ex2 = jnp.sum(xi * xi, axis=1, keepdims=True)
    mean_sq = jnp.maximum(ex2 * ((s * s) / _D), 0.0)
    d = jnp.clip(jnp.round(mean_sq), 1.0, 65535.0)

    q = jnp.zeros_like(d)
    for bit in range(7, -1, -1):
        c = q + float(1 << bit)
        q = jnp.where(c * c <= d, c, q)
    std = q + (d > q * q + q).astype(jnp.float32)
    inv = 1.0 / jnp.maximum(std, 1e-5)
    inv_ref[...] = inv

    y = w_ref[...] * ((xi * s) * inv)
    ymax_ref[...] = jnp.max(jnp.abs(y)).reshape(1, 1, 1)


def _dequant_body(xi8_ref, w_ref, inv_ref, sc_ref, o_ref):
    s_in = sc_ref[0, 0]
    s_out = sc_ref[0, 1]
    xi = xi8_ref[...].astype(jnp.float32)
    y = w_ref[...] * ((xi * s_in) * inv_ref[...])
    o_ref[...] = jnp.clip(jnp.round(y / s_out), -127.0, 127.0) * s_out


def kernel(x, weight):
    xf = x.reshape(_ROWS, _D)
    w2 = weight.reshape(1, _D)

    pmax = pl.pallas_call(
        _absmax_body,
        grid=(_NBLK,),
        in_specs=[pl.BlockSpec((_R, _D), lambda i: (i, 0))],
        out_specs=pl.BlockSpec((1, 1, 1), lambda i: (i, 0, 0)),
        out_shape=jax.ShapeDtypeStruct((_NBLK, 1, 1), jnp.float32),
        compiler_params=pltpu.CompilerParams(
            dimension_semantics=("arbitrary",)),
        name="armsnorm_absmax",
    )(xf)
    scale_in = jnp.maximum(jnp.max(pmax) / 127.0, 1e-8)

    xi8, inv, ymax = pl.pallas_call(
        _quant_stats_body,
        grid=(_NBLK,),
        in_specs=[
            pl.BlockSpec((_R, _D), lambda i: (i, 0)),
            pl.BlockSpec((1, _D), lambda i: (0, 0)),
            pl.BlockSpec(memory_space=pltpu.SMEM),
        ],
        out_specs=[
            pl.BlockSpec((_R, _D), lambda i: (i, 0)),
            pl.BlockSpec((_R, 1), lambda i: (i, 0)),
            pl.BlockSpec((1, 1, 1), lambda i: (i, 0, 0)),
        ],
        out_shape=[
            jax.ShapeDtypeStruct((_ROWS, _D), jnp.int8),
            jax.ShapeDtypeStruct((_ROWS, 1), jnp.float32),
            jax.ShapeDtypeStruct((_NBLK, 1, 1), jnp.float32),
        ],
        compiler_params=pltpu.CompilerParams(
            dimension_semantics=("arbitrary",)),
        name="armsnorm_quant_stats",
    )(xf, w2, scale_in.reshape(1, 1))
    scale_out = jnp.maximum(jnp.max(ymax) / 127.0, 1e-8)

    scales = jnp.concatenate(
        [scale_in.reshape(1, 1), scale_out.reshape(1, 1)], axis=1)
    yq = pl.pallas_call(
        _dequant_body,
        grid=(_NBLK,),
        in_specs=[
            pl.BlockSpec((_R, _D), lambda i: (i, 0)),
            pl.BlockSpec((1, _D), lambda i: (0, 0)),
            pl.BlockSpec((_R, 1), lambda i: (i, 0)),
            pl.BlockSpec(memory_space=pltpu.SMEM),
        ],
        out_specs=pl.BlockSpec((_R, _D), lambda i: (i, 0)),
        out_shape=jax.ShapeDtypeStruct((_ROWS, _D), jnp.float32),
        compiler_params=pltpu.CompilerParams(
            dimension_semantics=("arbitrary",)),
        name="armsnorm_dequant",
    )(xi8, w2, inv, scales)
    return yq.reshape(x.shape)
